# baseline (speedup 1.0000x reference)
import functools

import jax
import jax.numpy as jnp
from jax import lax
from jax.experimental import pallas as pl
from jax.experimental.pallas import tpu as pltpu
from jax.experimental.pallas import tpu_sc as plsc

B = 4
NPB = 16384
K = 16
M1, M2, M3 = 4096, 1024, 256


def _fps_body(m, n, bsz, px_ref, py_ref, pz_ref, idx_ref, qx_ref, qy_ref, qz_ref):
    c = n // 8
    mc = m // 8
    iota_g = (lax.broadcasted_iota(jnp.int32, (8, c), 0) * c
              + lax.broadcasted_iota(jnp.int32, (8, c), 1))
    iota_m = (lax.broadcasted_iota(jnp.int32, (8, mc), 0) * mc
              + lax.broadcasted_iota(jnp.int32, (8, mc), 1))
    pxs = [px_ref[b] for b in range(bsz)]
    pys = [py_ref[b] for b in range(bsz)]
    pzs = [pz_ref[b] for b in range(bsz)]

    init = []
    for b in range(bsz):
        x0 = pxs[b][0:1, 0:1]
        y0 = pys[b][0:1, 0:1]
        z0 = pzs[b][0:1, 0:1]
        dx = pxs[b] - x0
        dy = pys[b] - y0
        dz = pzs[b] - z0
        d = (dx * dx + dy * dy) + dz * dz
        hit0 = iota_m == 0
        init.append((d,
                     jnp.zeros((8, mc), jnp.int32),
                     jnp.where(hit0, x0, 0.0),
                     jnp.where(hit0, y0, 0.0),
                     jnp.where(hit0, z0, 0.0)))

    def red2(x, op):
        return op(op(x, axis=1, keepdims=True), axis=0, keepdims=True)

    def body(i, st):
        out = []
        for b in range(bsz):
            d, acc, ax, ay, az = st[b]
            mx = red2(d, jnp.max)
            nxt = red2(jnp.where(d == mx, iota_g, n), jnp.min)
            sel = iota_g == nxt
            cx = red2(jnp.where(sel, pxs[b], 0.0), jnp.sum)
            cy = red2(jnp.where(sel, pys[b], 0.0), jnp.sum)
            cz = red2(jnp.where(sel, pzs[b], 0.0), jnp.sum)
            hit = iota_m == i
            acc = jnp.where(hit, nxt, acc)
            ax = jnp.where(hit, cx, ax)
            ay = jnp.where(hit, cy, ay)
            az = jnp.where(hit, cz, az)
            ddx = pxs[b] - cx
            ddy = pys[b] - cy
            ddz = pzs[b] - cz
            dn = (ddx * ddx + ddy * ddy) + ddz * ddz
            out.append((jnp.minimum(d, dn), acc, ax, ay, az))
        return tuple(out)

    st = lax.fori_loop(1, m, body, tuple(init))
    for b in range(bsz):
        _, acc, ax, ay, az = st[b]
        idx_ref[b] = acc
        qx_ref[b] = ax
        qy_ref[b] = ay
        qz_ref[b] = az


def _fps_pallas(px, py, pz, m, interpret=False):
    bsz, n = px.shape
    p3 = lambda a: a.reshape(bsz, 8, n // 8)
    out = pl.pallas_call(
        functools.partial(_fps_body, m, n, bsz),
        out_shape=[
            jax.ShapeDtypeStruct((bsz, 8, m // 8), jnp.int32),
            jax.ShapeDtypeStruct((bsz, 8, m // 8), jnp.float32),
            jax.ShapeDtypeStruct((bsz, 8, m // 8), jnp.float32),
            jax.ShapeDtypeStruct((bsz, 8, m // 8), jnp.float32),
        ],
        interpret=interpret,
    )(p3(px), p3(py), p3(pz))
    return [a.reshape(bsz, m) for a in out]


def _knn_body(n, k, tq, q_ref, pt_ref, out_ref):
    b = pl.program_id(0)
    q = q_ref[0]
    pt = pt_ref[0]
    q2 = jnp.sum(q * q, axis=1, keepdims=True)
    p2 = jnp.sum(pt * pt, axis=0, keepdims=True)
    qp = jax.lax.dot_general(q, pt, (((1,), (0,)), ((), ())),
                             preferred_element_type=jnp.float32)
    d = q2 + p2 - 2.0 * qp
    iota_n = lax.broadcasted_iota(jnp.int32, (tq, n), 1)
    base = b * n
    bits = lax.bitcast_convert_type(d, jnp.int32)
    ikey = bits ^ ((bits >> 31) & jnp.int32(0x7FFFFFFF))
    key = (ikey & jnp.int32(~16383)) | iota_n
    sentinel = jnp.int32(0x7FFFFFFF)
    for r in range(k):
        mn = jnp.min(key, axis=1, keepdims=True)
        out_ref[0, :, r] = (mn[:, 0] & 16383) + base
        key = jnp.where(key == mn, sentinel, key)


def _knn_pallas(q, pT, k, interpret=False):
    bsz, m, _ = q.shape
    n = pT.shape[2]
    tq = min(256, m)
    grid = (bsz, m // tq)
    return pl.pallas_call(
        functools.partial(_knn_body, n, k, tq),
        grid=grid,
        in_specs=[
            pl.BlockSpec((1, tq, 3), lambda b, j: (b, j, 0)),
            pl.BlockSpec((1, 3, n), lambda b, j: (b, 0, 0)),
        ],
        out_specs=pl.BlockSpec((1, tq, k), lambda b, j: (b, j, 0)),
        out_shape=jax.ShapeDtypeStruct((bsz, m, k), jnp.int32),
        interpret=interpret,
    )(q, pT)


def _sc_gather(table, idx):
    d = table.shape[1]
    e = idx.shape[0]
    nw = 32
    per_w = e // nw
    chunk = 128
    nch = per_w // chunk
    mesh = plsc.VectorSubcoreMesh(core_axis_name="c", subcore_axis_name="s")

    @functools.partial(
        pl.kernel,
        out_type=jax.ShapeDtypeStruct((e, d), jnp.float32),
        mesh=mesh,
        scratch_types=[
            pltpu.VMEM((chunk,), jnp.int32),
            pltpu.VMEM((chunk, d), jnp.float32),
            pltpu.SemaphoreType.DMA,
        ],
        compiler_params=pltpu.CompilerParams(use_tc_tiling_on_sc=False),
    )
    def k(table_hbm, idx_hbm, out_hbm, idx_v, rows_v, sem):
        wid = lax.axis_index("s") * 2 + lax.axis_index("c")
        base = wid * per_w

        def body(t, carry):
            off = base + t * chunk
            pltpu.sync_copy(idx_hbm.at[pl.ds(off, chunk)], idx_v)
            pltpu.async_copy(table_hbm.at[idx_v], rows_v, sem).wait()
            pltpu.sync_copy(rows_v, out_hbm.at[pl.ds(off, chunk)])
            return carry

        lax.fori_loop(0, nch, body, 0)

    return k(table, idx)


def _conv_body(kk, tqc, g_ref, posc_ref, w1p_ref, b1_ref, w1r_ref, w2_ref,
               b2_ref, wsp_ref, bs_ref, wsr_ref, out_ref):
    g = g_ref[...]
    posc = posc_ref[...]
    hid = w1p_ref.shape[1]
    fout = w2_ref.shape[1]
    z1 = jnp.dot(g, w1p_ref[...], preferred_element_type=jnp.float32) + b1_ref[...]
    c1 = jnp.dot(posc, w1r_ref[...], preferred_element_type=jnp.float32)
    a1 = jax.nn.relu(z1.reshape(tqc, kk, hid) - c1.reshape(tqc, 1, hid))
    a1 = a1.reshape(tqc * kk, hid)
    z2 = (jnp.dot(a1, w2_ref[...], preferred_element_type=jnp.float32)
          + b2_ref[...]
          + jnp.dot(g, wsp_ref[...], preferred_element_type=jnp.float32)
          + bs_ref[...])
    cs = jnp.dot(posc, wsr_ref[...], preferred_element_type=jnp.float32)
    msg = jax.nn.relu(z2.reshape(tqc, kk, fout) - cs.reshape(tqc, 1, fout))
    out_ref[...] = jnp.max(msg, axis=1)


def _conv_pallas(g, posc, w1p, b1, w1r, w2, b2, wsp, bs, wsr, interpret=False):
    e, dpad = g.shape
    nc = posc.shape[0]
    kk = e // nc
    hid = w1p.shape[1]
    fout = w2.shape[1]
    tqc = min(256, nc)
    grid = (nc // tqc,)
    wspec = lambda a: pl.BlockSpec(a.shape, lambda i: (0,) * a.ndim)
    return pl.pallas_call(
        functools.partial(_conv_body, kk, tqc),
        grid=grid,
        in_specs=[
            pl.BlockSpec((tqc * kk, dpad), lambda i: (i, 0)),
            pl.BlockSpec((tqc, 3), lambda i: (i, 0)),
            wspec(w1p), wspec(b1), wspec(w1r), wspec(w2), wspec(b2),
            wspec(wsp), wspec(bs), wspec(wsr),
        ],
        out_specs=pl.BlockSpec((tqc, fout), lambda i: (i, 0)),
        out_shape=jax.ShapeDtypeStruct((nc, fout), jnp.float32),
        interpret=interpret,
    )(g, posc, w1p, b1, w1r, w2, b2, wsp, bs, wsr)


def _head_body(x3_ref, c_ref, fw1a_ref, fw1b_ref, fb1_ref, fw2_ref, fb2_ref,
               fwsa_ref, fwsb_ref, fbs_ref, muw_ref, mub_ref, lvw_ref,
               lvb_ref, mu_ref, lv_ref):
    x3 = x3_ref[...]
    m3 = x3.shape[0] // B
    yf = jnp.max(x3.reshape(B, m3, x3.shape[1]), axis=1)
    cc = c_ref[...]
    dot = lambda a, w: jnp.dot(a, w, preferred_element_type=jnp.float32)
    z1 = dot(yf, fw1a_ref[...]) + dot(cc, fw1b_ref[...]) + fb1_ref[...]
    a1 = jax.nn.relu(z1)
    z2 = (dot(a1, fw2_ref[...]) + fb2_ref[...]
          + dot(yf, fwsa_ref[...]) + dot(cc, fwsb_ref[...]) + fbs_ref[...])
    h = jax.nn.relu(z2)
    mu_ref[...] = dot(h, muw_ref[...]) + mub_ref[...]
    lv_ref[...] = dot(h, lvw_ref[...]) + lvb_ref[...]


def _head_pallas(x3, c, fw1a, fw1b, fb1, fw2, fb2, fwsa, fwsb, fbs,
                 muw, mub, lvw, lvb, interpret=False):
    lat = muw.shape[1]
    return pl.pallas_call(
        _head_body,
        out_shape=[jax.ShapeDtypeStruct((B, lat), jnp.float32),
                   jax.ShapeDtypeStruct((B, lat), jnp.float32)],
        interpret=interpret,
    )(x3, c, fw1a, fw1b, fb1, fw2, fb2, fwsa, fwsb, fbs, muw, mub, lvw, lvb)


def _knn_jax(query, points, k, chunk=256):
    p2 = jnp.sum(points**2, axis=1)
    m, d = query.shape
    qc = query.reshape(m // chunk, chunk, d)

    def f(q):
        d2 = jnp.sum(q**2, axis=1, keepdims=True) + p2[None, :] - 2.0 * (q @ points.T)
        return lax.top_k(-d2, k)[1]

    return lax.map(f, qc).reshape(m, k)


def _resmlp(h, w1, b1, w2, b2, ws, bs):
    return jax.nn.relu(h @ w1 + b1) @ w2 + b2 + (h @ ws + bs)


def kernel(y, query_pos, query_pos_batch, c,
           sa1_w1, sa1_b1, sa1_w2, sa1_b2, sa1_ws, sa1_bs,
           sa2_w1, sa2_b1, sa2_w2, sa2_b2, sa2_ws, sa2_bs,
           sa3_w1, sa3_b1, sa3_w2, sa3_b2, sa3_ws, sa3_bs,
           fin_w1, fin_b1, fin_w2, fin_b2, fin_ws, fin_bs,
           mu_w, mu_b, lv_w, lv_b):
    p1 = (sa1_w1, sa1_b1, sa1_w2, sa1_b2, sa1_ws, sa1_bs)
    p2 = (sa2_w1, sa2_b1, sa2_w2, sa2_b2, sa2_ws, sa2_bs)
    p3 = (sa3_w1, sa3_b1, sa3_w2, sa3_b2, sa3_ws, sa3_bs)

    pos = query_pos.reshape(B, NPB, 3)
    post = pos.transpose(0, 2, 1)
    xs = y.reshape(B, NPB, 3)

    def level(posT, m):
        idx, qx, qy, qz = _fps_pallas(posT[:, 0], posT[:, 1], posT[:, 2], m)
        posiT = jnp.stack([qx, qy, qz], axis=1)
        return idx, posiT

    idx1, pos1T = level(post, M1)
    idx2, pos2T = level(pos1T, M2)
    idx3, pos3T = level(pos2T, M3)
    q1 = pos1T.transpose(0, 2, 1)
    q2 = pos2T.transpose(0, 2, 1)
    q3 = pos3T.transpose(0, 2, 1)
    nbr1 = _knn_pallas(q1, post, K)
    nbr2 = _knn_pallas(q2, pos1T, K)
    nbr3 = _knn_pallas(q3, pos2T, K)

    def prep(w1, b1, w2, b2, ws, bs, f, dpad):
        w1p = jnp.pad(w1, ((0, dpad - f - 3), (0, 0)))
        wsp = jnp.pad(ws, ((0, dpad - f - 3), (0, 0)))
        return (w1p, b1.reshape(1, -1), w1[f:f + 3], w2, b2.reshape(1, -1),
                wsp, bs.reshape(1, -1), ws[f:f + 3])

    def level_conv(table, nbr, qc, params, f, dpad):
        g = _sc_gather(table, nbr.reshape(-1))
        posc = qc.reshape(-1, 3)
        return _conv_pallas(g, posc, *prep(*params, f, dpad))

    t1 = jnp.pad(jnp.concatenate([xs.reshape(B * NPB, 3),
                                  pos.reshape(B * NPB, 3)], axis=1),
                 ((0, 0), (0, 10)))
    x1 = level_conv(t1, nbr1, q1, p1, 3, 16)

    t2 = jnp.pad(jnp.concatenate([x1, q1.reshape(-1, 3)], axis=1),
                 ((0, 0), (0, 13)))
    x2 = level_conv(t2, nbr2, q2, p2, 64, 80)

    t3 = jnp.pad(jnp.concatenate([x2, q2.reshape(-1, 3)], axis=1),
                 ((0, 0), (0, 13)))
    x3 = level_conv(t3, nbr3, q3, p3, 128, 144)

    mu, lv = _head_pallas(
        x3, c, fin_w1[:256], fin_w1[256:], fin_b1.reshape(1, -1), fin_w2,
        fin_b2.reshape(1, -1), fin_ws[:256], fin_ws[256:],
        fin_bs.reshape(1, -1), mu_w, mu_b.reshape(1, -1), lv_w,
        lv_b.reshape(1, -1))
    return (mu, lv)


# speedup vs baseline: 1.0813x; 1.0813x over previous
import functools

import jax
import jax.numpy as jnp
from jax import lax
from jax.experimental import pallas as pl
from jax.experimental.pallas import tpu as pltpu
from jax.experimental.pallas import tpu_sc as plsc

B = 4
NPB = 16384
K = 16
M1, M2, M3 = 4096, 1024, 256


def _fps_body(m, n, bsz, px_ref, py_ref, pz_ref, idx_ref, qx_ref, qy_ref, qz_ref):
    c = n // 8
    mc = m // 8
    iota_g = (lax.broadcasted_iota(jnp.int32, (8, c), 0) * c
              + lax.broadcasted_iota(jnp.int32, (8, c), 1))
    iota_m = (lax.broadcasted_iota(jnp.int32, (8, mc), 0) * mc
              + lax.broadcasted_iota(jnp.int32, (8, mc), 1))
    pxs = [px_ref[b] for b in range(bsz)]
    pys = [py_ref[b] for b in range(bsz)]
    pzs = [pz_ref[b] for b in range(bsz)]

    init = []
    for b in range(bsz):
        x0 = pxs[b][0:1, 0:1]
        y0 = pys[b][0:1, 0:1]
        z0 = pzs[b][0:1, 0:1]
        dx = pxs[b] - x0
        dy = pys[b] - y0
        dz = pzs[b] - z0
        d = (dx * dx + dy * dy) + dz * dz
        hit0 = iota_m == 0
        init.append((d,
                     jnp.zeros((8, mc), jnp.int32),
                     jnp.where(hit0, x0, 0.0),
                     jnp.where(hit0, y0, 0.0),
                     jnp.where(hit0, z0, 0.0)))

    def red2(x, op):
        return op(op(x, axis=1, keepdims=True), axis=0, keepdims=True)

    def body(i, st):
        out = []
        for b in range(bsz):
            d, acc, ax, ay, az = st[b]
            mx = red2(d, jnp.max)
            nxt = red2(jnp.where(d == mx, iota_g, n), jnp.min)
            sel = iota_g == nxt
            cx = red2(jnp.where(sel, pxs[b], 0.0), jnp.sum)
            cy = red2(jnp.where(sel, pys[b], 0.0), jnp.sum)
            cz = red2(jnp.where(sel, pzs[b], 0.0), jnp.sum)
            hit = iota_m == i
            acc = jnp.where(hit, nxt, acc)
            ax = jnp.where(hit, cx, ax)
            ay = jnp.where(hit, cy, ay)
            az = jnp.where(hit, cz, az)
            ddx = pxs[b] - cx
            ddy = pys[b] - cy
            ddz = pzs[b] - cz
            dn = (ddx * ddx + ddy * ddy) + ddz * ddz
            out.append((jnp.minimum(d, dn), acc, ax, ay, az))
        return tuple(out)

    st = lax.fori_loop(1, m, body, tuple(init))
    for b in range(bsz):
        _, acc, ax, ay, az = st[b]
        idx_ref[b] = acc
        qx_ref[b] = ax
        qy_ref[b] = ay
        qz_ref[b] = az


def _fps_pallas(px, py, pz, m, interpret=False):
    bsz, n = px.shape
    p3 = lambda a: a.reshape(bsz, 8, n // 8)
    out = pl.pallas_call(
        functools.partial(_fps_body, m, n, bsz),
        out_shape=[
            jax.ShapeDtypeStruct((bsz, 8, m // 8), jnp.int32),
            jax.ShapeDtypeStruct((bsz, 8, m // 8), jnp.float32),
            jax.ShapeDtypeStruct((bsz, 8, m // 8), jnp.float32),
            jax.ShapeDtypeStruct((bsz, 8, m // 8), jnp.float32),
        ],
        interpret=interpret,
    )(p3(px), p3(py), p3(pz))
    return [a.reshape(bsz, m) for a in out]


def _knn_body(n, k, tq, q_ref, pt_ref, out_ref):
    b = pl.program_id(0)
    q = q_ref[0]
    pt = pt_ref[0]
    q2 = jnp.sum(q * q, axis=1, keepdims=True)
    p2 = jnp.sum(pt * pt, axis=0, keepdims=True)
    qp = jax.lax.dot_general(q, pt, (((1,), (0,)), ((), ())),
                             preferred_element_type=jnp.float32)
    d = q2 + p2 - 2.0 * qp
    iota_n = lax.broadcasted_iota(jnp.int32, (tq, n), 1)
    base = b * n
    bits = lax.bitcast_convert_type(d, jnp.int32)
    ikey = bits ^ ((bits >> 31) & jnp.int32(0x7FFFFFFF))
    key = (ikey & jnp.int32(~16383)) | iota_n
    sentinel = jnp.int32(0x7FFFFFFF)
    for r in range(k):
        mn = jnp.min(key, axis=1, keepdims=True)
        out_ref[0, :, r] = (mn[:, 0] & 16383) + base
        key = jnp.where(key == mn, sentinel, key)


def _knn_pallas(q, pT, k, interpret=False):
    bsz, m, _ = q.shape
    n = pT.shape[2]
    tq = min(128, m)
    grid = (bsz, m // tq)
    return pl.pallas_call(
        functools.partial(_knn_body, n, k, tq),
        grid=grid,
        in_specs=[
            pl.BlockSpec((1, tq, 3), lambda b, j: (b, j, 0)),
            pl.BlockSpec((1, 3, n), lambda b, j: (b, 0, 0)),
        ],
        out_specs=pl.BlockSpec((1, tq, k), lambda b, j: (b, j, 0)),
        out_shape=jax.ShapeDtypeStruct((bsz, m, k), jnp.int32),
        interpret=interpret,
    )(q, pT)


def _sc_gather(table, idx):
    d = table.shape[1]
    e = idx.shape[0]
    nw = 32
    per_w = e // nw
    chunk = 128
    nch = per_w // chunk
    mesh = plsc.VectorSubcoreMesh(core_axis_name="c", subcore_axis_name="s")

    @functools.partial(
        pl.kernel,
        out_type=jax.ShapeDtypeStruct((e, d), jnp.float32),
        mesh=mesh,
        scratch_types=[
            pltpu.VMEM((chunk,), jnp.int32),
            pltpu.VMEM((chunk, d), jnp.float32),
            pltpu.SemaphoreType.DMA,
        ],
        compiler_params=pltpu.CompilerParams(use_tc_tiling_on_sc=False),
    )
    def k(table_hbm, idx_hbm, out_hbm, idx_v, rows_v, sem):
        wid = lax.axis_index("s") * 2 + lax.axis_index("c")
        base = wid * per_w

        def body(t, carry):
            off = base + t * chunk
            pltpu.sync_copy(idx_hbm.at[pl.ds(off, chunk)], idx_v)
            pltpu.async_copy(table_hbm.at[idx_v], rows_v, sem).wait()
            pltpu.sync_copy(rows_v, out_hbm.at[pl.ds(off, chunk)])
            return carry

        lax.fori_loop(0, nch, body, 0)

    return k(table, idx)


def _conv_body(kk, tqc, g_ref, posc_ref, w1p_ref, b1_ref, w1r_ref, w2_ref,
               b2_ref, wsp_ref, bs_ref, wsr_ref, out_ref):
    g = g_ref[...]
    posc = posc_ref[...]
    hid = w1p_ref.shape[1]
    fout = w2_ref.shape[1]
    z1 = jnp.dot(g, w1p_ref[...], preferred_element_type=jnp.float32) + b1_ref[...]
    c1 = jnp.dot(posc, w1r_ref[...], preferred_element_type=jnp.float32)
    a1 = jax.nn.relu(z1.reshape(tqc, kk, hid) - c1.reshape(tqc, 1, hid))
    a1 = a1.reshape(tqc * kk, hid)
    z2 = (jnp.dot(a1, w2_ref[...], preferred_element_type=jnp.float32)
          + b2_ref[...]
          + jnp.dot(g, wsp_ref[...], preferred_element_type=jnp.float32)
          + bs_ref[...])
    cs = jnp.dot(posc, wsr_ref[...], preferred_element_type=jnp.float32)
    msg = jax.nn.relu(z2.reshape(tqc, kk, fout) - cs.reshape(tqc, 1, fout))
    out_ref[...] = jnp.max(msg, axis=1)


def _conv_pallas(g, posc, w1p, b1, w1r, w2, b2, wsp, bs, wsr, interpret=False):
    e, dpad = g.shape
    nc = posc.shape[0]
    kk = e // nc
    hid = w1p.shape[1]
    fout = w2.shape[1]
    tqc = min(256, nc)
    grid = (nc // tqc,)
    wspec = lambda a: pl.BlockSpec(a.shape, lambda i: (0,) * a.ndim)
    return pl.pallas_call(
        functools.partial(_conv_body, kk, tqc),
        grid=grid,
        in_specs=[
            pl.BlockSpec((tqc * kk, dpad), lambda i: (i, 0)),
            pl.BlockSpec((tqc, 3), lambda i: (i, 0)),
            wspec(w1p), wspec(b1), wspec(w1r), wspec(w2), wspec(b2),
            wspec(wsp), wspec(bs), wspec(wsr),
        ],
        out_specs=pl.BlockSpec((tqc, fout), lambda i: (i, 0)),
        out_shape=jax.ShapeDtypeStruct((nc, fout), jnp.float32),
        interpret=interpret,
    )(g, posc, w1p, b1, w1r, w2, b2, wsp, bs, wsr)


def _head_body(x3_ref, c_ref, fw1a_ref, fw1b_ref, fb1_ref, fw2_ref, fb2_ref,
               fwsa_ref, fwsb_ref, fbs_ref, muw_ref, mub_ref, lvw_ref,
               lvb_ref, mu_ref, lv_ref):
    x3 = x3_ref[...]
    m3 = x3.shape[0] // B
    yf = jnp.max(x3.reshape(B, m3, x3.shape[1]), axis=1)
    cc = c_ref[...]
    dot = lambda a, w: jnp.dot(a, w, preferred_element_type=jnp.float32)
    z1 = dot(yf, fw1a_ref[...]) + dot(cc, fw1b_ref[...]) + fb1_ref[...]
    a1 = jax.nn.relu(z1)
    z2 = (dot(a1, fw2_ref[...]) + fb2_ref[...]
          + dot(yf, fwsa_ref[...]) + dot(cc, fwsb_ref[...]) + fbs_ref[...])
    h = jax.nn.relu(z2)
    mu_ref[...] = dot(h, muw_ref[...]) + mub_ref[...]
    lv_ref[...] = dot(h, lvw_ref[...]) + lvb_ref[...]


def _head_pallas(x3, c, fw1a, fw1b, fb1, fw2, fb2, fwsa, fwsb, fbs,
                 muw, mub, lvw, lvb, interpret=False):
    lat = muw.shape[1]
    return pl.pallas_call(
        _head_body,
        out_shape=[jax.ShapeDtypeStruct((B, lat), jnp.float32),
                   jax.ShapeDtypeStruct((B, lat), jnp.float32)],
        interpret=interpret,
    )(x3, c, fw1a, fw1b, fb1, fw2, fb2, fwsa, fwsb, fbs, muw, mub, lvw, lvb)


def _knn_jax(query, points, k, chunk=256):
    p2 = jnp.sum(points**2, axis=1)
    m, d = query.shape
    qc = query.reshape(m // chunk, chunk, d)

    def f(q):
        d2 = jnp.sum(q**2, axis=1, keepdims=True) + p2[None, :] - 2.0 * (q @ points.T)
        return lax.top_k(-d2, k)[1]

    return lax.map(f, qc).reshape(m, k)


def _resmlp(h, w1, b1, w2, b2, ws, bs):
    return jax.nn.relu(h @ w1 + b1) @ w2 + b2 + (h @ ws + bs)


def kernel(y, query_pos, query_pos_batch, c,
           sa1_w1, sa1_b1, sa1_w2, sa1_b2, sa1_ws, sa1_bs,
           sa2_w1, sa2_b1, sa2_w2, sa2_b2, sa2_ws, sa2_bs,
           sa3_w1, sa3_b1, sa3_w2, sa3_b2, sa3_ws, sa3_bs,
           fin_w1, fin_b1, fin_w2, fin_b2, fin_ws, fin_bs,
           mu_w, mu_b, lv_w, lv_b):
    p1 = (sa1_w1, sa1_b1, sa1_w2, sa1_b2, sa1_ws, sa1_bs)
    p2 = (sa2_w1, sa2_b1, sa2_w2, sa2_b2, sa2_ws, sa2_bs)
    p3 = (sa3_w1, sa3_b1, sa3_w2, sa3_b2, sa3_ws, sa3_bs)

    pos = query_pos.reshape(B, NPB, 3)
    post = pos.transpose(0, 2, 1)
    xs = y.reshape(B, NPB, 3)

    def level(posT, m):
        idx, qx, qy, qz = _fps_pallas(posT[:, 0], posT[:, 1], posT[:, 2], m)
        posiT = jnp.stack([qx, qy, qz], axis=1)
        return idx, posiT

    idx1, pos1T = level(post, M1)
    idx2, pos2T = level(pos1T, M2)
    idx3, pos3T = level(pos2T, M3)
    q1 = pos1T.transpose(0, 2, 1)
    q2 = pos2T.transpose(0, 2, 1)
    q3 = pos3T.transpose(0, 2, 1)
    nbr1 = _knn_pallas(q1, post, K)
    nbr2 = _knn_pallas(q2, pos1T, K)
    nbr3 = _knn_pallas(q3, pos2T, K)

    def prep(w1, b1, w2, b2, ws, bs, f, dpad):
        w1p = jnp.pad(w1, ((0, dpad - f - 3), (0, 0)))
        wsp = jnp.pad(ws, ((0, dpad - f - 3), (0, 0)))
        return (w1p, b1.reshape(1, -1), w1[f:f + 3], w2, b2.reshape(1, -1),
                wsp, bs.reshape(1, -1), ws[f:f + 3])

    def level_conv(table, nbr, qc, params, f, dpad):
        g = _sc_gather(table, nbr.reshape(-1))
        posc = qc.reshape(-1, 3)
        return _conv_pallas(g, posc, *prep(*params, f, dpad))

    t1 = jnp.pad(jnp.concatenate([xs.reshape(B * NPB, 3),
                                  pos.reshape(B * NPB, 3)], axis=1),
                 ((0, 0), (0, 10)))
    x1 = level_conv(t1, nbr1, q1, p1, 3, 16)

    t2 = jnp.pad(jnp.concatenate([x1, q1.reshape(-1, 3)], axis=1),
                 ((0, 0), (0, 13)))
    x2 = level_conv(t2, nbr2, q2, p2, 64, 80)

    t3 = jnp.pad(jnp.concatenate([x2, q2.reshape(-1, 3)], axis=1),
                 ((0, 0), (0, 13)))
    x3 = level_conv(t3, nbr3, q3, p3, 128, 144)

    mu, lv = _head_pallas(
        x3, c, fin_w1[:256], fin_w1[256:], fin_b1.reshape(1, -1), fin_w2,
        fin_b2.reshape(1, -1), fin_ws[:256], fin_ws[256:],
        fin_bs.reshape(1, -1), mu_w, mu_b.reshape(1, -1), lv_w,
        lv_b.reshape(1, -1))
    return (mu, lv)
